# baseline (speedup 1.0000x reference)
import functools

import jax
import jax.numpy as jnp
from jax import lax
from jax.experimental import pallas as pl
from jax.experimental.pallas import tpu as pltpu
from jax.experimental.pallas import tpu_sc as plsc

N = 10000
D = 128
NPAD = 10240
NC = 2
NS = 16
NW = NC * NS
ROWS_PER_S = NPAD // NS
RB = ROWS_PER_S // 128


def _sc_mesh():
    return plsc.VectorSubcoreMesh(core_axis_name="c", subcore_axis_name="s")


def _wid():
    return lax.axis_index("s") * NC + lax.axis_index("c")


def _deg_call(dst2d, ew2d, z1d, chunks):
    @functools.partial(
        pl.kernel,
        out_type=jax.ShapeDtypeStruct((NC, NPAD), jnp.float32),
        mesh=_sc_mesh(),
        compiler_params=pltpu.CompilerParams(
            needs_layout_passes=False, use_tc_tiling_on_sc=False),
        scratch_types=[
            pltpu.VMEM((chunks, 128), jnp.int32),
            pltpu.VMEM((chunks, 128), jnp.float32),
            pltpu.VMEM_SHARED((NPAD,), jnp.float32),
        ],
    )
    def deg_kernel(dst_hbm, ew_hbm, z_hbm, degp_hbm, dst_v, ew_v, deg_sh):
        c = lax.axis_index("c")
        s = lax.axis_index("s")
        wid = _wid()
        pltpu.sync_copy(z_hbm, deg_sh.at[pl.ds(s * ROWS_PER_S, ROWS_PER_S)])
        pltpu.sync_copy(dst_hbm.at[wid], dst_v)
        pltpu.sync_copy(ew_hbm.at[wid], ew_v)
        plsc.subcore_barrier()

        def body(g, carry):
            pltpu.sync_copy(ew_v.at[g], deg_sh.at[dst_v.at[g]], add=True)
            return carry

        lax.fori_loop(0, chunks, body, 0)
        plsc.subcore_barrier()
        sl = pl.ds(s * ROWS_PER_S, ROWS_PER_S)
        pltpu.sync_copy(deg_sh.at[sl], degp_hbm.at[c, sl])

    return deg_kernel(dst2d, ew2d, z1d)


def _agg_call(h, src2d, dst2d, norm2d, zblk, chunks):
    G = next(g for g in range(16, 1, -2) if chunks % g == 0)
    ngroups = chunks // G
    HD = D // 2

    @functools.partial(
        pl.kernel,
        out_type=jax.ShapeDtypeStruct((NC, NPAD, D), jnp.float32),
        mesh=_sc_mesh(),
        compiler_params=pltpu.CompilerParams(
            needs_layout_passes=False, use_tc_tiling_on_sc=False),
        scratch_types=[
            pltpu.VMEM((2, G, 128), jnp.int32),
            pltpu.VMEM((2, G, 128), jnp.int32),
            pltpu.VMEM((2, G, 128), jnp.float32),
            pltpu.VMEM((128, HD), jnp.float32),
            pltpu.VMEM((128, HD), jnp.float32),
            pltpu.VMEM_SHARED((NPAD, HD), jnp.float32),
            pltpu.VMEM_SHARED((NPAD, HD), jnp.float32),
            pltpu.SemaphoreType.DMA,
            pltpu.SemaphoreType.DMA,
            pltpu.SemaphoreType.DMA,
            pltpu.SemaphoreType.DMA,
        ],
    )
    def agg_kernel(h_hbm, src_hbm, dst_hbm, norm_hbm, z_hbm, accp_hbm,
                   src_v, dst_v, nrm_v, rows0, rows1, h_sh, acc_sh,
                   sg0, sg1, ss0, ss1):
        c = lax.axis_index("c")
        s = lax.axis_index("s")
        wid = _wid()
        rows = (rows0, rows1)
        sg = (sg0, sg1)
        ss = (ss0, ss1)

        def wait_gather(b):
            pltpu.make_async_copy(h_sh.at[pl.ds(0, 128)], rows[b],
                                  sg[b]).wait()

        def wait_scatter(b):
            pltpu.make_async_copy(rows[b], acc_sh.at[pl.ds(0, 128)],
                                  ss[b]).wait()

        def scale(gp, g, b):
            def scale16(t, carry):
                n16 = nrm_v[gp, g, pl.ds(t * 16, 16)]
                for j in range(16):
                    e = t * 16 + j
                    bc = jnp.broadcast_to(n16[j], (16,))
                    for cg in range(HD // 16):
                        cols = pl.ds(cg * 16, 16)
                        rows[b][e, cols] = rows[b][e, cols] * bc
                return carry

            lax.fori_loop(0, 8, scale16, 0, unroll=2)

        for half in range(2):
            csl = pl.ds(half * HD, HD)
            msl = pl.ds(s * ROWS_PER_S, ROWS_PER_S)
            pltpu.sync_copy(h_hbm.at[msl, csl], h_sh.at[msl])
            for k in range(RB):
                pltpu.sync_copy(z_hbm,
                                acc_sh.at[pl.ds((s * RB + k) * 128, 128)])
            g0 = pl.ds(0, G)
            pltpu.sync_copy(src_hbm.at[wid, g0], src_v.at[0])
            pltpu.sync_copy(dst_hbm.at[wid, g0], dst_v.at[0])
            pltpu.sync_copy(norm_hbm.at[wid, g0], nrm_v.at[0])
            plsc.subcore_barrier()
            pltpu.async_copy(h_sh.at[src_v.at[0, 0]], rows0, sg0)

            def step(gg, gp, gq, k, b, carry):
                gc = gg * G + 2 * k + b
                nxt = 1 - b

                @pl.when(gc + 1 < chunks)
                def _():
                    if b == 0:
                        @pl.when(k >= 1)
                        def _():
                            wait_scatter(nxt)
                        pltpu.async_copy(h_sh.at[src_v.at[gp, 2 * k + 1]],
                                         rows[nxt], sg[nxt])
                    else:
                        wait_scatter(nxt)

                        @pl.when(2 * k + 2 == G)
                        def _():
                            pltpu.async_copy(h_sh.at[src_v.at[gq, 0]],
                                             rows[nxt], sg[nxt])

                        @pl.when(2 * k + 2 < G)
                        def _():
                            pltpu.async_copy(
                                h_sh.at[src_v.at[gp, 2 * k + 2]],
                                rows[nxt], sg[nxt])

                wait_gather(b)
                scale(gp, 2 * k + b, b)
                pltpu.async_copy(rows[b],
                                 acc_sh.at[dst_v.at[gp, 2 * k + b]],
                                 ss[b], add=True)
                return carry

            def group(gg, carry):
                gp = gg % 2
                gq = 1 - gp

                @pl.when(gg >= 1)
                def _():
                    wait_scatter(1)

                @pl.when(gg + 1 < ngroups)
                def _():
                    nsl = pl.ds((gg + 1) * G, G)
                    pltpu.sync_copy(src_hbm.at[wid, nsl], src_v.at[gq])
                    pltpu.sync_copy(dst_hbm.at[wid, nsl], dst_v.at[gq])
                    pltpu.sync_copy(norm_hbm.at[wid, nsl], nrm_v.at[gq])

                def pair(k, carry1):
                    carry1 = step(gg, gp, gq, k, 0, carry1)
                    carry1 = step(gg, gp, gq, k, 1, carry1)
                    return carry1

                return lax.fori_loop(0, G // 2, pair, carry)

            lax.fori_loop(0, ngroups, group, 0)
            wait_scatter(0)
            wait_scatter(1)
            plsc.subcore_barrier()
            for k in range(RB):
                rsl = pl.ds((s * RB + k) * 128, 128)
                pltpu.sync_copy(acc_sh.at[rsl], accp_hbm.at[c, rsl, csl])

    return agg_kernel(h, src2d, dst2d, norm2d, zblk)


def _mm1_call(degp, xp, W):
    grid = (NPAD // 1024,)

    def body(deg_ref, x_ref, w_ref, dinv_ref, h_ref):
        deg = deg_ref[0] + deg_ref[1]
        dinv = jnp.where(deg > 0, lax.rsqrt(deg), 0.0)
        dinv_ref[...] = dinv
        h = jnp.dot(x_ref[...], w_ref[...],
                    preferred_element_type=jnp.float32,
                    precision=lax.Precision.HIGHEST)
        h_ref[...] = dinv * h

    return pl.pallas_call(
        body,
        grid=grid,
        in_specs=[
            pl.BlockSpec((2, 1024, 1), lambda i: (0, i, 0)),
            pl.BlockSpec((1024, D), lambda i: (i, 0)),
            pl.BlockSpec((D, D), lambda i: (0, 0)),
        ],
        out_specs=[
            pl.BlockSpec((1024, 1), lambda i: (i, 0)),
            pl.BlockSpec((1024, D), lambda i: (i, 0)),
        ],
        out_shape=[
            jax.ShapeDtypeStruct((NPAD, 1), jnp.float32),
            jax.ShapeDtypeStruct((NPAD, D), jnp.float32),
        ],
    )(degp.reshape(2, NPAD, 1), xp, W)


def _mm2_call(accp, dinv, b, a, W):
    grid = (NPAD // 1024,)

    def body(acc_ref, dinv_ref, b_ref, a_ref, w_ref, h_ref):
        dv = dinv_ref[...]
        x = dv * (acc_ref[0] + acc_ref[1]) + b_ref[...]
        x = jnp.where(x >= 0, x, a_ref[...] * x)
        h = jnp.dot(x, w_ref[...],
                    preferred_element_type=jnp.float32,
                    precision=lax.Precision.HIGHEST)
        h_ref[...] = dv * h

    return pl.pallas_call(
        body,
        grid=grid,
        in_specs=[
            pl.BlockSpec((2, 1024, D), lambda i: (0, i, 0)),
            pl.BlockSpec((1024, 1), lambda i: (i, 0)),
            pl.BlockSpec((1, D), lambda i: (0, 0)),
            pl.BlockSpec((1, D), lambda i: (0, 0)),
            pl.BlockSpec((D, D), lambda i: (0, 0)),
        ],
        out_specs=pl.BlockSpec((1024, D), lambda i: (i, 0)),
        out_shape=jax.ShapeDtypeStruct((NPAD, D), jnp.float32),
    )(accp, dinv, b.reshape(1, D), a.reshape(1, D), W)


def _final_call(accp, dinv, b, a):
    grid = (NPAD // 1024,)

    def body(acc_ref, dinv_ref, b_ref, a_ref, o_ref):
        dv = dinv_ref[...]
        x = dv * (acc_ref[0] + acc_ref[1]) + b_ref[...]
        o_ref[...] = jnp.where(x >= 0, x, a_ref[...] * x)

    return pl.pallas_call(
        body,
        grid=grid,
        in_specs=[
            pl.BlockSpec((2, 1024, D), lambda i: (0, i, 0)),
            pl.BlockSpec((1024, 1), lambda i: (i, 0)),
            pl.BlockSpec((1, D), lambda i: (0, 0)),
            pl.BlockSpec((1, D), lambda i: (0, 0)),
        ],
        out_specs=pl.BlockSpec((1024, D), lambda i: (i, 0)),
        out_shape=jax.ShapeDtypeStruct((NPAD, D), jnp.float32),
    )(accp, dinv, b.reshape(1, D), a.reshape(1, D))


def kernel(features, edge_index, edge_weight, W1, b1, a1, W2, b2, a2):
    E = edge_index.shape[1]
    EF = E + NPAD
    chunks = ((EF + NW * 128 - 1) // (NW * 128) + 11) // 12 * 12
    EPAD = chunks * NW * 128

    xp = jnp.pad(features, ((0, NPAD - N), (0, 0)))
    loop = jnp.arange(NPAD, dtype=jnp.int32)
    tail = EPAD - EF
    pad_idx = jnp.arange(tail, dtype=jnp.int32) % NPAD
    srcf = jnp.concatenate(
        [edge_index[0], loop, pad_idx]).reshape(NW, -1, 128)
    dstf = jnp.concatenate(
        [edge_index[1], loop, pad_idx]).reshape(NW, -1, 128)
    ewf = jnp.concatenate(
        [edge_weight, jnp.ones((NPAD,), jnp.float32),
         jnp.zeros((tail,), jnp.float32)]).reshape(NW, -1, 128)
    z1d = jnp.zeros((ROWS_PER_S,), jnp.float32)
    zblk = jnp.zeros((128, D // 2), jnp.float32)

    degp = _deg_call(dstf, ewf, z1d, chunks)
    dinv, h1 = _mm1_call(degp, xp, W1)
    accp1 = _agg_call(h1, srcf, dstf, ewf, zblk, chunks)
    h2 = _mm2_call(accp1, dinv, b1, a1, W2)
    accp2 = _agg_call(h2, srcf, dstf, ewf, zblk, chunks)
    out = _final_call(accp2, dinv, b2, a2)
    return out[:N]


# speedup vs baseline: 1.1990x; 1.1990x over previous
import functools

import jax
import jax.numpy as jnp
from jax import lax
from jax.experimental import pallas as pl
from jax.experimental.pallas import tpu as pltpu
from jax.experimental.pallas import tpu_sc as plsc

N = 10000
D = 128
NPAD = 10240
NC = 2
NS = 16
NW = NC * NS
ROWS_PER_S = NPAD // NS
RB = ROWS_PER_S // 128


def _sc_mesh():
    return plsc.VectorSubcoreMesh(core_axis_name="c", subcore_axis_name="s")


def _wid():
    return lax.axis_index("s") * NC + lax.axis_index("c")


def _deg_call(dst2d, ew2d, z1d, chunks):
    @functools.partial(
        pl.kernel,
        out_type=jax.ShapeDtypeStruct((NC, NPAD), jnp.float32),
        mesh=_sc_mesh(),
        compiler_params=pltpu.CompilerParams(
            needs_layout_passes=False, use_tc_tiling_on_sc=False),
        scratch_types=[
            pltpu.VMEM((chunks, 128), jnp.int32),
            pltpu.VMEM((chunks, 128), jnp.float32),
            pltpu.VMEM_SHARED((NPAD,), jnp.float32),
        ],
    )
    def deg_kernel(dst_hbm, ew_hbm, z_hbm, degp_hbm, dst_v, ew_v, deg_sh):
        c = lax.axis_index("c")
        s = lax.axis_index("s")
        wid = _wid()
        pltpu.sync_copy(z_hbm, deg_sh.at[pl.ds(s * ROWS_PER_S, ROWS_PER_S)])
        pltpu.sync_copy(dst_hbm.at[wid], dst_v)
        pltpu.sync_copy(ew_hbm.at[wid], ew_v)
        plsc.subcore_barrier()

        def body(g, carry):
            pltpu.sync_copy(ew_v.at[g], deg_sh.at[dst_v.at[g]], add=True)
            return carry

        lax.fori_loop(0, chunks, body, 0)
        plsc.subcore_barrier()
        sl = pl.ds(s * ROWS_PER_S, ROWS_PER_S)
        pltpu.sync_copy(deg_sh.at[sl], degp_hbm.at[c, sl])

    return deg_kernel(dst2d, ew2d, z1d)


def _agg_call(h, src2d, dst2d, norm2d, zblk, chunks):
    G = next(g for g in range(16, 1, -2) if chunks % g == 0)
    ngroups = chunks // G

    @functools.partial(
        pl.kernel,
        out_type=jax.ShapeDtypeStruct((NC, NPAD, D), jnp.float32),
        mesh=_sc_mesh(),
        compiler_params=pltpu.CompilerParams(
            needs_layout_passes=False, use_tc_tiling_on_sc=False),
        scratch_types=[
            pltpu.VMEM((2, G, 128), jnp.int32),
            pltpu.VMEM((2, G, 128), jnp.int32),
            pltpu.VMEM((2, G, 128), jnp.float32),
            pltpu.VMEM((128, D), jnp.float32),
            pltpu.VMEM((128, D), jnp.float32),
            pltpu.VMEM_SHARED((NPAD, D), jnp.float32),
            pltpu.SemaphoreType.DMA,
            pltpu.SemaphoreType.DMA,
            pltpu.SemaphoreType.DMA,
            pltpu.SemaphoreType.DMA,
        ],
    )
    def agg_kernel(h_hbm, src_hbm, dst_hbm, norm_hbm, z_hbm, accp_hbm,
                   src_v, dst_v, nrm_v, rows0, rows1, acc_sh,
                   sg0, sg1, ss0, ss1):
        c = lax.axis_index("c")
        s = lax.axis_index("s")
        wid = _wid()
        rows = (rows0, rows1)
        sg = (sg0, sg1)
        ss = (ss0, ss1)

        for k in range(RB):
            pltpu.sync_copy(z_hbm,
                            acc_sh.at[pl.ds((s * RB + k) * 128, 128)])
        g0 = pl.ds(0, G)
        pltpu.sync_copy(src_hbm.at[wid, g0], src_v.at[0])
        pltpu.sync_copy(dst_hbm.at[wid, g0], dst_v.at[0])
        pltpu.sync_copy(norm_hbm.at[wid, g0], nrm_v.at[0])
        plsc.subcore_barrier()
        pltpu.async_copy(h_hbm.at[src_v.at[0, 0]], rows0, sg0)

        def wait_gather(b):
            pltpu.make_async_copy(h_hbm, rows[b], sg[b]).wait()

        def wait_scatter(b):
            pltpu.make_async_copy(rows[b], acc_sh.at[pl.ds(0, 128)],
                                  ss[b]).wait()

        def scale(gp, g, b):
            def scale16(t, carry):
                n16 = nrm_v[gp, g, pl.ds(t * 16, 16)]
                for j in range(16):
                    e = t * 16 + j
                    bc = jnp.broadcast_to(n16[j], (16,))
                    for cg in range(8):
                        cols = pl.ds(cg * 16, 16)
                        rows[b][e, cols] = rows[b][e, cols] * bc
                return carry

            lax.fori_loop(0, 8, scale16, 0, unroll=2)

        def step(gg, gp, gq, k, b, carry):
            gc = gg * G + 2 * k + b
            nxt = 1 - b

            @pl.when(gc + 1 < chunks)
            def _():
                if b == 0:
                    @pl.when(k >= 1)
                    def _():
                        wait_scatter(nxt)
                    pltpu.async_copy(h_hbm.at[src_v.at[gp, 2 * k + 1]],
                                     rows[nxt], sg[nxt])
                else:
                    wait_scatter(nxt)

                    @pl.when(2 * k + 2 == G)
                    def _():
                        pltpu.async_copy(h_hbm.at[src_v.at[gq, 0]],
                                         rows[nxt], sg[nxt])

                    @pl.when(2 * k + 2 < G)
                    def _():
                        pltpu.async_copy(h_hbm.at[src_v.at[gp, 2 * k + 2]],
                                         rows[nxt], sg[nxt])

            wait_gather(b)
            scale(gp, 2 * k + b, b)
            pltpu.async_copy(rows[b], acc_sh.at[dst_v.at[gp, 2 * k + b]],
                             ss[b], add=True)
            return carry

        def group(gg, carry):
            gp = gg % 2
            gq = 1 - gp

            @pl.when(gg >= 1)
            def _():
                wait_scatter(1)

            @pl.when(gg + 1 < ngroups)
            def _():
                nsl = pl.ds((gg + 1) * G, G)
                pltpu.sync_copy(src_hbm.at[wid, nsl], src_v.at[gq])
                pltpu.sync_copy(dst_hbm.at[wid, nsl], dst_v.at[gq])
                pltpu.sync_copy(norm_hbm.at[wid, nsl], nrm_v.at[gq])

            def pair(k, carry1):
                carry1 = step(gg, gp, gq, k, 0, carry1)
                carry1 = step(gg, gp, gq, k, 1, carry1)
                return carry1

            return lax.fori_loop(0, G // 2, pair, carry)

        lax.fori_loop(0, ngroups, group, 0)
        wait_scatter(0)
        wait_scatter(1)
        plsc.subcore_barrier()
        for k in range(RB):
            rsl = pl.ds((s * RB + k) * 128, 128)
            pltpu.sync_copy(acc_sh.at[rsl], accp_hbm.at[c, rsl])

    return agg_kernel(h, src2d, dst2d, norm2d, zblk)


def _mm1_call(degp, xp, W):
    grid = (NPAD // 1024,)

    def body(deg_ref, x_ref, w_ref, dinv_ref, h_ref):
        deg = deg_ref[0] + deg_ref[1]
        dinv = jnp.where(deg > 0, lax.rsqrt(deg), 0.0)
        dinv_ref[...] = dinv
        h = jnp.dot(x_ref[...], w_ref[...],
                    preferred_element_type=jnp.float32,
                    precision=lax.Precision.HIGHEST)
        h_ref[...] = dinv * h

    return pl.pallas_call(
        body,
        grid=grid,
        in_specs=[
            pl.BlockSpec((2, 1024, 1), lambda i: (0, i, 0)),
            pl.BlockSpec((1024, D), lambda i: (i, 0)),
            pl.BlockSpec((D, D), lambda i: (0, 0)),
        ],
        out_specs=[
            pl.BlockSpec((1024, 1), lambda i: (i, 0)),
            pl.BlockSpec((1024, D), lambda i: (i, 0)),
        ],
        out_shape=[
            jax.ShapeDtypeStruct((NPAD, 1), jnp.float32),
            jax.ShapeDtypeStruct((NPAD, D), jnp.float32),
        ],
    )(degp.reshape(2, NPAD, 1), xp, W)


def _mm2_call(accp, dinv, b, a, W):
    grid = (NPAD // 1024,)

    def body(acc_ref, dinv_ref, b_ref, a_ref, w_ref, h_ref):
        dv = dinv_ref[...]
        x = dv * (acc_ref[0] + acc_ref[1]) + b_ref[...]
        x = jnp.where(x >= 0, x, a_ref[...] * x)
        h = jnp.dot(x, w_ref[...],
                    preferred_element_type=jnp.float32,
                    precision=lax.Precision.HIGHEST)
        h_ref[...] = dv * h

    return pl.pallas_call(
        body,
        grid=grid,
        in_specs=[
            pl.BlockSpec((2, 1024, D), lambda i: (0, i, 0)),
            pl.BlockSpec((1024, 1), lambda i: (i, 0)),
            pl.BlockSpec((1, D), lambda i: (0, 0)),
            pl.BlockSpec((1, D), lambda i: (0, 0)),
            pl.BlockSpec((D, D), lambda i: (0, 0)),
        ],
        out_specs=pl.BlockSpec((1024, D), lambda i: (i, 0)),
        out_shape=jax.ShapeDtypeStruct((NPAD, D), jnp.float32),
    )(accp, dinv, b.reshape(1, D), a.reshape(1, D), W)


def _final_call(accp, dinv, b, a):
    grid = (NPAD // 1024,)

    def body(acc_ref, dinv_ref, b_ref, a_ref, o_ref):
        dv = dinv_ref[...]
        x = dv * (acc_ref[0] + acc_ref[1]) + b_ref[...]
        o_ref[...] = jnp.where(x >= 0, x, a_ref[...] * x)

    return pl.pallas_call(
        body,
        grid=grid,
        in_specs=[
            pl.BlockSpec((2, 1024, D), lambda i: (0, i, 0)),
            pl.BlockSpec((1024, 1), lambda i: (i, 0)),
            pl.BlockSpec((1, D), lambda i: (0, 0)),
            pl.BlockSpec((1, D), lambda i: (0, 0)),
        ],
        out_specs=pl.BlockSpec((1024, D), lambda i: (i, 0)),
        out_shape=jax.ShapeDtypeStruct((NPAD, D), jnp.float32),
    )(accp, dinv, b.reshape(1, D), a.reshape(1, D))


def kernel(features, edge_index, edge_weight, W1, b1, a1, W2, b2, a2):
    E = edge_index.shape[1]
    EF = E + NPAD
    chunks = ((EF + NW * 128 - 1) // (NW * 128) + 11) // 12 * 12
    EPAD = chunks * NW * 128

    xp = jnp.pad(features, ((0, NPAD - N), (0, 0)))
    loop = jnp.arange(NPAD, dtype=jnp.int32)
    tail = EPAD - EF
    pad_idx = jnp.arange(tail, dtype=jnp.int32) % NPAD
    srcf = jnp.concatenate(
        [edge_index[0], loop, pad_idx]).reshape(NW, -1, 128)
    dstf = jnp.concatenate(
        [edge_index[1], loop, pad_idx]).reshape(NW, -1, 128)
    ewf = jnp.concatenate(
        [edge_weight, jnp.ones((NPAD,), jnp.float32),
         jnp.zeros((tail,), jnp.float32)]).reshape(NW, -1, 128)
    z1d = jnp.zeros((ROWS_PER_S,), jnp.float32)
    zblk = jnp.zeros((128, D), jnp.float32)

    degp = _deg_call(dstf, ewf, z1d, chunks)
    dinv, h1 = _mm1_call(degp, xp, W1)
    accp1 = _agg_call(h1, srcf, dstf, ewf, zblk, chunks)
    h2 = _mm2_call(accp1, dinv, b1, a1, W2)
    accp2 = _agg_call(h2, srcf, dstf, ewf, zblk, chunks)
    out = _final_call(accp2, dinv, b2, a2)
    return out[:N]
